# baseline (speedup 1.0000x reference)
import functools

import jax
import jax.numpy as jnp
from jax import lax
from jax.experimental import pallas as pl
from jax.experimental.pallas import tpu as pltpu
from jax.experimental.pallas import tpu_sc as plsc

N_NODES = 10000
N_EDGES = 320000
DIM = 128
HALF = DIM // 2
COUT = 40

NCORES = 2
NSUB = 16
NW = NCORES * NSUB
CHD = 128
CH = 256
KD = 86
KC = 86
ECAP = NW * KD * CHD
NPAD = 10240
ROWS_PER_TILE = NPAD // NSUB
NBUF = 2
GLEAD = 1

_mesh = plsc.VectorSubcoreMesh(core_axis_name="c", subcore_axis_name="s")
_sc_params = pltpu.CompilerParams(
    needs_layout_passes=False, use_tc_tiling_on_sc=False
)


@functools.partial(
    pl.kernel,
    out_type=jax.ShapeDtypeStruct((NCORES, NPAD), jnp.float32),
    mesh=_mesh,
    compiler_params=_sc_params,
    scratch_types=[
        pltpu.VMEM((KD, CHD), jnp.int32),
        pltpu.VMEM((KD, CHD), jnp.float32),
        pltpu.VMEM((ROWS_PER_TILE,), jnp.float32),
        pltpu.VMEM_SHARED((NPAD,), jnp.float32),
    ],
)
def _deg_kernel(col_hbm, ew_hbm, out_hbm, col_v, ew_v, zbuf, deg_sh):
    c = lax.axis_index("c")
    s = lax.axis_index("s")
    w = c * NSUB + s
    pltpu.sync_copy(col_hbm.at[w], col_v)
    pltpu.sync_copy(ew_hbm.at[w], ew_v)
    zero = jnp.zeros((16,), jnp.float32)
    for i in range(ROWS_PER_TILE // 16):
        zbuf[pl.ds(i * 16, 16)] = zero
    pltpu.sync_copy(zbuf, deg_sh.at[pl.ds(s * ROWS_PER_TILE, ROWS_PER_TILE)])
    plsc.subcore_barrier()

    def body(j, carry):
        pltpu.sync_copy(ew_v.at[j], deg_sh.at[col_v.at[j]], add=True)
        return carry

    lax.fori_loop(0, KD, body, 0)
    plsc.subcore_barrier()
    pltpu.sync_copy(
        deg_sh.at[pl.ds(s * ROWS_PER_TILE, ROWS_PER_TILE)],
        out_hbm.at[c, pl.ds(s * ROWS_PER_TILE, ROWS_PER_TILE)],
    )


def _mm_body(x_ref, wi_ref, bi_ref, wg_ref, degp_ref, h2a_ref, h2b_ref,
             dinv_ref):
    h = jnp.dot(x_ref[...], wi_ref[...], preferred_element_type=jnp.float32)
    h = jnp.maximum(h + bi_ref[...], 0.0)
    h2 = jnp.dot(h, wg_ref[...], preferred_element_type=jnp.float32)
    h2a_ref[...] = h2[:, :HALF]
    h2b_ref[...] = h2[:, HALF:]
    d = degp_ref[0, :] + degp_ref[1, :]
    dinv_ref[...] = jnp.where(d > 0.0, lax.rsqrt(d), 0.0)


_mm_call = pl.pallas_call(
    _mm_body,
    out_shape=(
        jax.ShapeDtypeStruct((N_NODES, HALF), jnp.float32),
        jax.ShapeDtypeStruct((N_NODES, HALF), jnp.float32),
        jax.ShapeDtypeStruct((NPAD,), jnp.float32),
    ),
)


@functools.partial(
    pl.kernel,
    out_type=jax.ShapeDtypeStruct((NCORES, NPAD, HALF), jnp.float32),
    mesh=_mesh,
    compiler_params=_sc_params,
    scratch_types=[
        pltpu.VMEM((KC, CH), jnp.int32),
        pltpu.VMEM((KC, CH), jnp.int32),
        pltpu.VMEM((NBUF, CH), jnp.float32),
        pltpu.VMEM((NBUF, CH), jnp.float32),
        pltpu.VMEM((NBUF, CH), jnp.float32),
        pltpu.VMEM((NBUF, CH, HALF), jnp.float32),
        pltpu.VMEM_SHARED((NPAD, HALF), jnp.float32),
    ] + [pltpu.SemaphoreType.DMA] * (2 * NBUF),
)
def _conv_kernel(row_hbm, col_hbm, ew_hbm, dinv_hbm, h2a_hbm, h2b_hbm,
                 out_hbm, row_v, col_v, ewbuf, drbuf, dcbuf, rbuf,
                 acc_sh, *sems):
    gsems = sems[:NBUF]
    ssems = sems[NBUF:]
    c = lax.axis_index("c")
    s = lax.axis_index("s")
    pltpu.sync_copy(row_hbm.at[s], row_v)
    pltpu.sync_copy(col_hbm.at[s], col_v)

    zero = jnp.zeros((16,), jnp.float32)

    def zbody(i, carry):
        for q in range(HALF // 16):
            rbuf[0, i, pl.ds(q * 16, 16)] = zero
        return carry

    lax.fori_loop(0, CH, zbody, 0)
    for q in range(ROWS_PER_TILE // CH):
        pltpu.sync_copy(
            rbuf.at[0], acc_sh.at[pl.ds(s * ROWS_PER_TILE + q * CH, CH)]
        )
    rem = ROWS_PER_TILE % CH
    if rem:
        pltpu.sync_copy(
            rbuf.at[0, pl.ds(0, rem)],
            acc_sh.at[pl.ds(s * ROWS_PER_TILE + (ROWS_PER_TILE // CH) * CH, rem)],
        )
    plsc.subcore_barrier()

    def start_gather(j, b):
        @pl.when(c == 0)
        def _():
            pltpu.async_copy(h2a_hbm.at[row_v.at[j]], rbuf.at[b], gsems[b])

        @pl.when(c == 1)
        def _():
            pltpu.async_copy(h2b_hbm.at[row_v.at[j]], rbuf.at[b], gsems[b])

        pltpu.async_copy(dinv_hbm.at[row_v.at[j]], drbuf.at[b], gsems[b])
        pltpu.async_copy(dinv_hbm.at[col_v.at[j]], dcbuf.at[b], gsems[b])
        pltpu.async_copy(ew_hbm.at[s, j], ewbuf.at[b], gsems[b])

    def wait_gather(j, b):
        @pl.when(c == 0)
        def _():
            pltpu.make_async_copy(
                h2a_hbm.at[row_v.at[j]], rbuf.at[b], gsems[b]
            ).wait()

        @pl.when(c == 1)
        def _():
            pltpu.make_async_copy(
                h2b_hbm.at[row_v.at[j]], rbuf.at[b], gsems[b]
            ).wait()

        pltpu.make_async_copy(
            dinv_hbm.at[row_v.at[j]], drbuf.at[b], gsems[b]
        ).wait()
        pltpu.make_async_copy(
            dinv_hbm.at[col_v.at[j]], dcbuf.at[b], gsems[b]
        ).wait()
        pltpu.make_async_copy(
            ew_hbm.at[s, j], ewbuf.at[b], gsems[b]
        ).wait()

    def start_scatter(j, b):
        pltpu.async_copy(
            rbuf.at[b], acc_sh.at[col_v.at[j]], ssems[b], add=True
        )

    def wait_scatter(j, b):
        pltpu.make_async_copy(
            rbuf.at[b], acc_sh.at[col_v.at[j]], ssems[b]
        ).wait()

    def scale(j, b):
        def sbody(g, carry):
            sl = pl.ds(g * 16, 16)
            n16 = drbuf[b, sl] * ewbuf[b, sl] * dcbuf[b, sl]
            for l in range(16):
                sc = n16[l]
                e = g * 16 + l
                for q in range(HALF // 16):
                    rbuf[b, e, pl.ds(q * 16, 16)] = (
                        rbuf[b, e, pl.ds(q * 16, 16)] * sc
                    )
            return carry

        lax.fori_loop(0, CH // 16, sbody, 0)

    for i in range(GLEAD):
        start_gather(i, i)

    def round_body(u, carry):
        base = u * NBUF
        for i in range(NBUF):
            j = base + i
            bn = (i + GLEAD) % NBUF

            @pl.when(j >= NBUF - GLEAD)
            def _():
                wait_scatter(j + GLEAD - NBUF, bn)

            @pl.when(j + GLEAD < KC)
            def _():
                start_gather(j + GLEAD, bn)

            wait_gather(j, i)
            scale(j, i)
            start_scatter(j, i)
        return carry

    lax.fori_loop(0, KC // NBUF, round_body, 0)
    for i in range(NBUF - GLEAD):
        j = KC - (NBUF - GLEAD) + i
        wait_scatter(j, j % NBUF)

    plsc.subcore_barrier()
    pltpu.sync_copy(
        acc_sh.at[pl.ds(s * ROWS_PER_TILE, ROWS_PER_TILE)],
        out_hbm.at[c, pl.ds(s * ROWS_PER_TILE, ROWS_PER_TILE)],
    )


def _out_body(s_ref, bg_ref, wo_ref, bo_ref, o_ref):
    ta = jnp.maximum(s_ref[0, :N_NODES, :] + bg_ref[:HALF], 0.0)
    tb = jnp.maximum(s_ref[1, :N_NODES, :] + bg_ref[HALF:], 0.0)
    o_ref[...] = (
        jnp.dot(ta, wo_ref[:HALF, :], preferred_element_type=jnp.float32)
        + jnp.dot(tb, wo_ref[HALF:, :], preferred_element_type=jnp.float32)
        + bo_ref[...]
    )


_out_call = pl.pallas_call(
    _out_body,
    out_shape=jax.ShapeDtypeStruct((N_NODES, COUT), jnp.float32),
)


@jax.jit
def kernel(x, edge_index, edge_weight, W_in, b_in, W_gcn, b_gcn, W_out, b_out):
    row = edge_index[0]
    col = edge_index[1]
    loop = jnp.arange(N_NODES, dtype=jnp.int32)
    npad = ECAP - (N_EDGES + N_NODES)
    rows_all = jnp.concatenate([row, loop, jnp.zeros((npad,), jnp.int32)])
    cols_all = jnp.concatenate([col, loop, jnp.full((npad,), N_NODES, jnp.int32)])
    ew_all = jnp.concatenate(
        [edge_weight, jnp.ones((N_NODES,), jnp.float32),
         jnp.zeros((npad,), jnp.float32)]
    )

    degp = _deg_kernel(
        cols_all.reshape(NW, KD, CHD), ew_all.reshape(NW, KD, CHD)
    )
    h2a, h2b, dinv = _mm_call(x, W_in, b_in, W_gcn, degp)
    s_acc = _conv_kernel(
        rows_all.reshape(NSUB, KC, CH),
        cols_all.reshape(NSUB, KC, CH),
        ew_all.reshape(NSUB, KC, CH),
        dinv, h2a, h2b,
    )
    return _out_call(s_acc, b_gcn, W_out, b_out)


# speedup vs baseline: 1.0110x; 1.0110x over previous
import functools

import jax
import jax.numpy as jnp
from jax import lax
from jax.experimental import pallas as pl
from jax.experimental.pallas import tpu as pltpu
from jax.experimental.pallas import tpu_sc as plsc

N_NODES = 10000
N_EDGES = 320000
DIM = 128
HALF = DIM // 2
COUT = 40

NCORES = 2
NSUB = 16
NW = NCORES * NSUB
CHD = 128
CH = 128
KD = 85
KC = 170
ECAP = NW * KD * CHD
NPAD = 10240
ROWS_PER_TILE = NPAD // NSUB
NBUF = 5
GLEAD = 2

_mesh = plsc.VectorSubcoreMesh(core_axis_name="c", subcore_axis_name="s")
_sc_params = pltpu.CompilerParams(
    needs_layout_passes=False, use_tc_tiling_on_sc=False
)


@functools.partial(
    pl.kernel,
    out_type=jax.ShapeDtypeStruct((NCORES, NPAD), jnp.float32),
    mesh=_mesh,
    compiler_params=_sc_params,
    scratch_types=[
        pltpu.VMEM((KD, CHD), jnp.int32),
        pltpu.VMEM((KD, CHD), jnp.float32),
        pltpu.VMEM((ROWS_PER_TILE,), jnp.float32),
        pltpu.VMEM_SHARED((NPAD,), jnp.float32),
    ],
)
def _deg_kernel(col_hbm, ew_hbm, out_hbm, col_v, ew_v, zbuf, deg_sh):
    c = lax.axis_index("c")
    s = lax.axis_index("s")
    w = c * NSUB + s
    pltpu.sync_copy(col_hbm.at[w], col_v)
    pltpu.sync_copy(ew_hbm.at[w], ew_v)
    zero = jnp.zeros((16,), jnp.float32)
    for i in range(ROWS_PER_TILE // 16):
        zbuf[pl.ds(i * 16, 16)] = zero
    pltpu.sync_copy(zbuf, deg_sh.at[pl.ds(s * ROWS_PER_TILE, ROWS_PER_TILE)])
    plsc.subcore_barrier()

    def body(j, carry):
        pltpu.sync_copy(ew_v.at[j], deg_sh.at[col_v.at[j]], add=True)
        return carry

    lax.fori_loop(0, KD, body, 0)
    plsc.subcore_barrier()
    pltpu.sync_copy(
        deg_sh.at[pl.ds(s * ROWS_PER_TILE, ROWS_PER_TILE)],
        out_hbm.at[c, pl.ds(s * ROWS_PER_TILE, ROWS_PER_TILE)],
    )


def _mm_body(x_ref, wi_ref, bi_ref, wg_ref, degp_ref, h2a_ref, h2b_ref,
             dinv_ref):
    h = jnp.dot(x_ref[...], wi_ref[...], preferred_element_type=jnp.float32)
    h = jnp.maximum(h + bi_ref[...], 0.0)
    h2 = jnp.dot(h, wg_ref[...], preferred_element_type=jnp.float32)
    h2a_ref[...] = h2[:, :HALF]
    h2b_ref[...] = h2[:, HALF:]
    d = degp_ref[0, :] + degp_ref[1, :]
    dinv_ref[...] = jnp.where(d > 0.0, lax.rsqrt(d), 0.0)


_mm_call = pl.pallas_call(
    _mm_body,
    out_shape=(
        jax.ShapeDtypeStruct((N_NODES, HALF), jnp.float32),
        jax.ShapeDtypeStruct((N_NODES, HALF), jnp.float32),
        jax.ShapeDtypeStruct((NPAD,), jnp.float32),
    ),
)


@functools.partial(
    pl.kernel,
    out_type=jax.ShapeDtypeStruct((NCORES, NPAD, HALF), jnp.float32),
    mesh=_mesh,
    compiler_params=_sc_params,
    scratch_types=[
        pltpu.VMEM((KC, CH), jnp.int32),
        pltpu.VMEM((KC, CH), jnp.int32),
        pltpu.VMEM((NBUF, CH), jnp.float32),
        pltpu.VMEM((NBUF, CH), jnp.float32),
        pltpu.VMEM((NBUF, CH), jnp.float32),
        pltpu.VMEM((NBUF, CH, HALF), jnp.float32),
        pltpu.VMEM_SHARED((NPAD, HALF), jnp.float32),
    ] + [pltpu.SemaphoreType.DMA] * (2 * NBUF),
)
def _conv_kernel(row_hbm, col_hbm, ew_hbm, dinv_hbm, h2a_hbm, h2b_hbm,
                 out_hbm, row_v, col_v, ewbuf, drbuf, dcbuf, rbuf,
                 acc_sh, *sems):
    gsems = sems[:NBUF]
    ssems = sems[NBUF:]
    c = lax.axis_index("c")
    s = lax.axis_index("s")
    pltpu.sync_copy(row_hbm.at[s], row_v)
    pltpu.sync_copy(col_hbm.at[s], col_v)

    zero = jnp.zeros((16,), jnp.float32)

    def zbody(i, carry):
        for q in range(HALF // 16):
            rbuf[0, i, pl.ds(q * 16, 16)] = zero
        return carry

    lax.fori_loop(0, CH, zbody, 0)
    for q in range(ROWS_PER_TILE // CH):
        pltpu.sync_copy(
            rbuf.at[0], acc_sh.at[pl.ds(s * ROWS_PER_TILE + q * CH, CH)]
        )
    rem = ROWS_PER_TILE % CH
    if rem:
        pltpu.sync_copy(
            rbuf.at[0, pl.ds(0, rem)],
            acc_sh.at[pl.ds(s * ROWS_PER_TILE + (ROWS_PER_TILE // CH) * CH, rem)],
        )
    plsc.subcore_barrier()

    def start_gather(j, b):
        @pl.when(c == 0)
        def _():
            pltpu.async_copy(h2a_hbm.at[row_v.at[j]], rbuf.at[b], gsems[b])

        @pl.when(c == 1)
        def _():
            pltpu.async_copy(h2b_hbm.at[row_v.at[j]], rbuf.at[b], gsems[b])

        pltpu.async_copy(dinv_hbm.at[row_v.at[j]], drbuf.at[b], gsems[b])
        pltpu.async_copy(dinv_hbm.at[col_v.at[j]], dcbuf.at[b], gsems[b])
        pltpu.async_copy(ew_hbm.at[s, j], ewbuf.at[b], gsems[b])

    def wait_gather(j, b):
        @pl.when(c == 0)
        def _():
            pltpu.make_async_copy(
                h2a_hbm.at[row_v.at[j]], rbuf.at[b], gsems[b]
            ).wait()

        @pl.when(c == 1)
        def _():
            pltpu.make_async_copy(
                h2b_hbm.at[row_v.at[j]], rbuf.at[b], gsems[b]
            ).wait()

        pltpu.make_async_copy(
            dinv_hbm.at[row_v.at[j]], drbuf.at[b], gsems[b]
        ).wait()
        pltpu.make_async_copy(
            dinv_hbm.at[col_v.at[j]], dcbuf.at[b], gsems[b]
        ).wait()
        pltpu.make_async_copy(
            ew_hbm.at[s, j], ewbuf.at[b], gsems[b]
        ).wait()

    def start_scatter(j, b):
        pltpu.async_copy(
            rbuf.at[b], acc_sh.at[col_v.at[j]], ssems[b], add=True
        )

    def wait_scatter(j, b):
        pltpu.make_async_copy(
            rbuf.at[b], acc_sh.at[col_v.at[j]], ssems[b]
        ).wait()

    def scale(j, b):
        def sbody(g, carry):
            sl = pl.ds(g * 16, 16)
            n16 = drbuf[b, sl] * ewbuf[b, sl] * dcbuf[b, sl]
            for l in range(16):
                sc = n16[l]
                e = g * 16 + l
                for q in range(HALF // 16):
                    rbuf[b, e, pl.ds(q * 16, 16)] = (
                        rbuf[b, e, pl.ds(q * 16, 16)] * sc
                    )
            return carry

        lax.fori_loop(0, CH // 16, sbody, 0)

    for i in range(GLEAD):
        start_gather(i, i)

    def round_body(u, carry):
        base = u * NBUF
        for i in range(NBUF):
            j = base + i
            bn = (i + GLEAD) % NBUF

            @pl.when(j >= NBUF - GLEAD)
            def _():
                wait_scatter(j + GLEAD - NBUF, bn)

            @pl.when(j + GLEAD < KC)
            def _():
                start_gather(j + GLEAD, bn)

            wait_gather(j, i)
            scale(j, i)
            start_scatter(j, i)
        return carry

    lax.fori_loop(0, KC // NBUF, round_body, 0)
    for i in range(NBUF - GLEAD):
        j = KC - (NBUF - GLEAD) + i
        wait_scatter(j, j % NBUF)

    plsc.subcore_barrier()
    pltpu.sync_copy(
        acc_sh.at[pl.ds(s * ROWS_PER_TILE, ROWS_PER_TILE)],
        out_hbm.at[c, pl.ds(s * ROWS_PER_TILE, ROWS_PER_TILE)],
    )


def _out_body(s_ref, bg_ref, wo_ref, bo_ref, o_ref):
    ta = jnp.maximum(s_ref[0, :N_NODES, :] + bg_ref[:HALF], 0.0)
    tb = jnp.maximum(s_ref[1, :N_NODES, :] + bg_ref[HALF:], 0.0)
    o_ref[...] = (
        jnp.dot(ta, wo_ref[:HALF, :], preferred_element_type=jnp.float32)
        + jnp.dot(tb, wo_ref[HALF:, :], preferred_element_type=jnp.float32)
        + bo_ref[...]
    )


_out_call = pl.pallas_call(
    _out_body,
    out_shape=jax.ShapeDtypeStruct((N_NODES, COUT), jnp.float32),
)


@jax.jit
def kernel(x, edge_index, edge_weight, W_in, b_in, W_gcn, b_gcn, W_out, b_out):
    row = edge_index[0]
    col = edge_index[1]
    loop = jnp.arange(N_NODES, dtype=jnp.int32)
    npad = ECAP - (N_EDGES + N_NODES)
    rows_all = jnp.concatenate([row, loop, jnp.zeros((npad,), jnp.int32)])
    cols_all = jnp.concatenate([col, loop, jnp.full((npad,), N_NODES, jnp.int32)])
    ew_all = jnp.concatenate(
        [edge_weight, jnp.ones((N_NODES,), jnp.float32),
         jnp.zeros((npad,), jnp.float32)]
    )

    degp = _deg_kernel(
        cols_all.reshape(NW, KD, CHD), ew_all.reshape(NW, KD, CHD)
    )
    h2a, h2b, dinv = _mm_call(x, W_in, b_in, W_gcn, degp)
    s_acc = _conv_kernel(
        rows_all.reshape(NSUB, KC, CH),
        cols_all.reshape(NSUB, KC, CH),
        ew_all.reshape(NSUB, KC, CH),
        dinv, h2a, h2b,
    )
    return _out_call(s_acc, b_gcn, W_out, b_out)


# speedup vs baseline: 1.2934x; 1.2793x over previous
import functools

import jax
import jax.numpy as jnp
from jax import lax
from jax.experimental import pallas as pl
from jax.experimental.pallas import tpu as pltpu
from jax.experimental.pallas import tpu_sc as plsc

N_NODES = 10000
N_EDGES = 320000
DIM = 128
HALF = DIM // 2
COUT = 40

NCORES = 2
NSUB = 16
NW = NCORES * NSUB
CHD = 128
CH = 128
KD = 84
KC = 168
ECAP = NW * KD * CHD
NPAD = 10240
ROWS_PER_TILE = NPAD // NSUB
NBUF = 4
GLEAD = 2

_mesh = plsc.VectorSubcoreMesh(core_axis_name="c", subcore_axis_name="s")
_sc_params = pltpu.CompilerParams(
    needs_layout_passes=False, use_tc_tiling_on_sc=False
)


@functools.partial(
    pl.kernel,
    out_type=jax.ShapeDtypeStruct((NCORES, NPAD), jnp.float32),
    mesh=_mesh,
    compiler_params=_sc_params,
    scratch_types=[
        pltpu.VMEM((KD, CHD), jnp.int32),
        pltpu.VMEM((KD, CHD), jnp.float32),
        pltpu.VMEM((ROWS_PER_TILE,), jnp.float32),
        pltpu.VMEM_SHARED((NPAD,), jnp.float32),
    ],
)
def _deg_kernel(col_hbm, ew_hbm, out_hbm, col_v, ew_v, zbuf, deg_sh):
    c = lax.axis_index("c")
    s = lax.axis_index("s")
    w = c * NSUB + s
    pltpu.sync_copy(col_hbm.at[w], col_v)
    pltpu.sync_copy(ew_hbm.at[w], ew_v)
    zero = jnp.zeros((16,), jnp.float32)
    for i in range(ROWS_PER_TILE // 16):
        zbuf[pl.ds(i * 16, 16)] = zero
    pltpu.sync_copy(zbuf, deg_sh.at[pl.ds(s * ROWS_PER_TILE, ROWS_PER_TILE)])
    plsc.subcore_barrier()

    def body(j, carry):
        pltpu.sync_copy(ew_v.at[j], deg_sh.at[col_v.at[j]], add=True)
        return carry

    lax.fori_loop(0, KD, body, 0)
    plsc.subcore_barrier()
    pltpu.sync_copy(
        deg_sh.at[pl.ds(s * ROWS_PER_TILE, ROWS_PER_TILE)],
        out_hbm.at[c, pl.ds(s * ROWS_PER_TILE, ROWS_PER_TILE)],
    )


def _mm_body(x_ref, wi_ref, bi_ref, wg_ref, degp_ref, h2a_ref, h2b_ref,
             dinv_ref):
    h = jnp.dot(x_ref[...], wi_ref[...], preferred_element_type=jnp.float32)
    h = jnp.maximum(h + bi_ref[...], 0.0)
    h2 = jnp.dot(h, wg_ref[...], preferred_element_type=jnp.float32)
    h2a_ref[...] = h2[:, :HALF]
    h2b_ref[...] = h2[:, HALF:]
    d = degp_ref[0, :] + degp_ref[1, :]
    dinv_ref[...] = jnp.where(d > 0.0, lax.rsqrt(d), 0.0)


_mm_call = pl.pallas_call(
    _mm_body,
    out_shape=(
        jax.ShapeDtypeStruct((N_NODES, HALF), jnp.float32),
        jax.ShapeDtypeStruct((N_NODES, HALF), jnp.float32),
        jax.ShapeDtypeStruct((NPAD,), jnp.float32),
    ),
)


@functools.partial(
    pl.kernel,
    out_type=jax.ShapeDtypeStruct((NCORES, NPAD, HALF), jnp.float32),
    mesh=_mesh,
    compiler_params=_sc_params,
    scratch_types=[
        pltpu.VMEM((KC, CH), jnp.int32),
        pltpu.VMEM((KC, CH), jnp.int32),
        pltpu.VMEM((NBUF, CH), jnp.float32),
        pltpu.VMEM((NBUF, CH), jnp.float32),
        pltpu.VMEM((NBUF, CH), jnp.float32),
        pltpu.VMEM((NBUF, CH, HALF), jnp.float32),
        pltpu.VMEM_SHARED((NPAD, HALF), jnp.float32),
    ] + [pltpu.SemaphoreType.DMA] * (2 * NBUF),
)
def _conv_kernel(row_hbm, col_hbm, ew_hbm, dinv_hbm, h2a_hbm, h2b_hbm,
                 out_hbm, row_v, col_v, ewbuf, drbuf, dcbuf, rbuf,
                 acc_sh, *sems):
    gsems = sems[:NBUF]
    ssems = sems[NBUF:]
    c = lax.axis_index("c")
    s = lax.axis_index("s")
    pltpu.sync_copy(row_hbm.at[s], row_v)
    pltpu.sync_copy(col_hbm.at[s], col_v)

    zero = jnp.zeros((16,), jnp.float32)

    def zbody(i, carry):
        for q in range(HALF // 16):
            rbuf[0, i, pl.ds(q * 16, 16)] = zero
        return carry

    lax.fori_loop(0, CH, zbody, 0)
    for q in range(ROWS_PER_TILE // CH):
        pltpu.sync_copy(
            rbuf.at[0], acc_sh.at[pl.ds(s * ROWS_PER_TILE + q * CH, CH)]
        )
    rem = ROWS_PER_TILE % CH
    if rem:
        pltpu.sync_copy(
            rbuf.at[0, pl.ds(0, rem)],
            acc_sh.at[pl.ds(s * ROWS_PER_TILE + (ROWS_PER_TILE // CH) * CH, rem)],
        )
    plsc.subcore_barrier()

    def start_gather(j, b):
        @pl.when(c == 0)
        def _():
            pltpu.async_copy(h2a_hbm.at[row_v.at[j]], rbuf.at[b], gsems[b])

        @pl.when(c == 1)
        def _():
            pltpu.async_copy(h2b_hbm.at[row_v.at[j]], rbuf.at[b], gsems[b])

        pltpu.async_copy(dinv_hbm.at[row_v.at[j]], drbuf.at[b], gsems[b])
        pltpu.async_copy(dinv_hbm.at[col_v.at[j]], dcbuf.at[b], gsems[b])
        pltpu.async_copy(ew_hbm.at[s, j], ewbuf.at[b], gsems[b])

    def wait_gather(j, b):
        @pl.when(c == 0)
        def _():
            pltpu.make_async_copy(
                h2a_hbm.at[row_v.at[j]], rbuf.at[b], gsems[b]
            ).wait()

        @pl.when(c == 1)
        def _():
            pltpu.make_async_copy(
                h2b_hbm.at[row_v.at[j]], rbuf.at[b], gsems[b]
            ).wait()

        pltpu.make_async_copy(
            dinv_hbm.at[row_v.at[j]], drbuf.at[b], gsems[b]
        ).wait()
        pltpu.make_async_copy(
            dinv_hbm.at[col_v.at[j]], dcbuf.at[b], gsems[b]
        ).wait()
        pltpu.make_async_copy(
            ew_hbm.at[s, j], ewbuf.at[b], gsems[b]
        ).wait()

    def start_scatter(j, b):
        pltpu.async_copy(
            rbuf.at[b], acc_sh.at[col_v.at[j]], ssems[b], add=True
        )

    def wait_scatter(j, b):
        pltpu.make_async_copy(
            rbuf.at[b], acc_sh.at[col_v.at[j]], ssems[b]
        ).wait()

    def scale(j, b):
        def sbody(g, carry):
            sl = pl.ds(g * 16, 16)
            n16 = drbuf[b, sl] * ewbuf[b, sl] * dcbuf[b, sl]
            for l in range(16):
                sc = n16[l]
                e = g * 16 + l
                for q in range(HALF // 16):
                    rbuf[b, e, pl.ds(q * 16, 16)] = (
                        rbuf[b, e, pl.ds(q * 16, 16)] * sc
                    )
            return carry

        lax.fori_loop(0, CH // 16, sbody, 0)

    for i in range(GLEAD):
        start_gather(i, i)

    def round_body(u, carry):
        base = u * NBUF
        for i in range(NBUF):
            j = base + i
            bn = (i + GLEAD) % NBUF

            @pl.when(j >= NBUF - GLEAD)
            def _():
                wait_scatter(j + GLEAD - NBUF, bn)

            @pl.when(j + GLEAD < KC)
            def _():
                start_gather(j + GLEAD, bn)

            wait_gather(j, i)
            scale(j, i)
            start_scatter(j, i)
        return carry

    lax.fori_loop(0, KC // NBUF, round_body, 0)
    for i in range(NBUF - GLEAD):
        j = KC - (NBUF - GLEAD) + i
        wait_scatter(j, j % NBUF)

    plsc.subcore_barrier()
    pltpu.sync_copy(
        acc_sh.at[pl.ds(s * ROWS_PER_TILE, ROWS_PER_TILE)],
        out_hbm.at[c, pl.ds(s * ROWS_PER_TILE, ROWS_PER_TILE)],
    )


def _out_body(s_ref, bg_ref, wo_ref, bo_ref, o_ref):
    ta = jnp.maximum(s_ref[0, :N_NODES, :] + bg_ref[:HALF], 0.0)
    tb = jnp.maximum(s_ref[1, :N_NODES, :] + bg_ref[HALF:], 0.0)
    o_ref[...] = (
        jnp.dot(ta, wo_ref[:HALF, :], preferred_element_type=jnp.float32)
        + jnp.dot(tb, wo_ref[HALF:, :], preferred_element_type=jnp.float32)
        + bo_ref[...]
    )


_out_call = pl.pallas_call(
    _out_body,
    out_shape=jax.ShapeDtypeStruct((N_NODES, COUT), jnp.float32),
)


@jax.jit
def kernel(x, edge_index, edge_weight, W_in, b_in, W_gcn, b_gcn, W_out, b_out):
    row = edge_index[0]
    col = edge_index[1]
    loop = jnp.arange(N_NODES, dtype=jnp.int32)
    npad = ECAP - (N_EDGES + N_NODES)
    rows_all = jnp.concatenate([row, loop, jnp.zeros((npad,), jnp.int32)])
    cols_all = jnp.concatenate([col, loop, jnp.full((npad,), N_NODES, jnp.int32)])
    ew_all = jnp.concatenate(
        [edge_weight, jnp.ones((N_NODES,), jnp.float32),
         jnp.zeros((npad,), jnp.float32)]
    )

    degp = _deg_kernel(
        cols_all.reshape(NW, KD, CHD), ew_all.reshape(NW, KD, CHD)
    )
    h2a, h2b, dinv = _mm_call(x, W_in, b_in, W_gcn, degp)
    s_acc = _conv_kernel(
        rows_all.reshape(NSUB, KC, CH),
        cols_all.reshape(NSUB, KC, CH),
        ew_all.reshape(NSUB, KC, CH),
        dinv, h2a, h2b,
    )
    return _out_call(s_acc, b_gcn, W_out, b_out)
